# Initial kernel scaffold; baseline (speedup 1.0000x reference)
import jax
import jax.numpy as jnp
from jax.experimental import pallas as pl


def kernel(user_id, book_title, user_genre_cat_1, user_genre_cat_2, user_genre_cat_3, user_genre_cat_4, book_genre_cat_1, book_genre_cat_2, book_genre_cat_3, book_genre_cat_4, user_table, book_table, ug_table_1, ug_table_2, ug_table_3, ug_table_4, bg_table_1, bg_table_2, bg_table_3, bg_table_4, W1, b1, W2, b2, W3, b3, W4, b4):
    raise NotImplementedError("write your pallas kernel here")


# speedup vs baseline: 1.3564x; 1.3564x over previous
import functools

import jax
import jax.numpy as jnp
from jax import lax
from jax.experimental import pallas as pl
from jax.experimental.pallas import tpu as pltpu
from jax.experimental.pallas import tpu_sc as plsc

NC = 2
NS = 16
NW = NC * NS
LANES = 16

B = 16384
BPW = B // NW
CH = 128
NCHUNK = BPW // CH
EMB = 32
GEMB = 4
GROWS = 1001
NGT = 8
GFLAT = NGT * GROWS * GEMB


def _sc_gather(uidx, bidx, gidx, gflat, user_table, book_table):
    mesh = plsc.VectorSubcoreMesh(core_axis_name="c", subcore_axis_name="s")

    @functools.partial(
        pl.kernel,
        out_type=(
            jax.ShapeDtypeStruct((B, EMB), jnp.float32),
            jax.ShapeDtypeStruct((B, EMB), jnp.float32),
            jax.ShapeDtypeStruct((B, NGT * GEMB), jnp.float32),
        ),
        mesh=mesh,
        compiler_params=pltpu.CompilerParams(
            needs_layout_passes=False, use_tc_tiling_on_sc=False),
        scratch_types=(
            pltpu.VMEM((NCHUNK, CH), jnp.int32),
            pltpu.VMEM((NCHUNK, CH), jnp.int32),
            pltpu.VMEM((NGT, BPW), jnp.int32),
            pltpu.VMEM((GFLAT,), jnp.float32),
            pltpu.VMEM((BPW, EMB), jnp.float32),
            pltpu.VMEM((BPW, EMB), jnp.float32),
            pltpu.VMEM((BPW, NGT * GEMB), jnp.float32),
            pltpu.SemaphoreType.DMA,
        ),
    )
    def k(uidx_hbm, bidx_hbm, gidx_hbm, gflat_hbm, utab_hbm, btab_hbm,
          out_u, out_b, out_g,
          uidx_v, bidx_v, gidx_v, gtab_v, urows, brows, grows, sem):
        wid = lax.axis_index("s") * NC + lax.axis_index("c")
        base = wid * BPW

        pltpu.sync_copy(uidx_hbm.at[wid], uidx_v)
        pltpu.sync_copy(bidx_hbm.at[wid], bidx_v)

        copies = []
        for j in range(NCHUNK):
            copies.append(pltpu.async_copy(
                utab_hbm.at[uidx_v.at[j]], urows.at[pl.ds(j * CH, CH)], sem))
            copies.append(pltpu.async_copy(
                btab_hbm.at[bidx_v.at[j]], brows.at[pl.ds(j * CH, CH)], sem))

        pltpu.sync_copy(gidx_hbm.at[wid], gidx_v)
        pltpu.sync_copy(gflat_hbm, gtab_v)

        iota = lax.iota(jnp.int32, LANES)

        def vec_body(v, carry):
            row0 = v * LANES
            rows_idx = row0 + iota
            for t in range(NGT):
                ids = gidx_v.at[t][pl.ds(row0, LANES)]
                flat = ids * GEMB + (t * GROWS * GEMB)
                for c in range(GEMB):
                    vals = plsc.load_gather(gtab_v, [flat + c])
                    col = jnp.full((LANES,), t * GEMB + c, jnp.int32)
                    plsc.store_scatter(grows, [rows_idx, col], vals)
            return carry

        lax.fori_loop(0, BPW // LANES, vec_body, 0)

        for c in copies:
            c.wait()

        pltpu.sync_copy(urows, out_u.at[pl.ds(base, BPW)])
        pltpu.sync_copy(brows, out_b.at[pl.ds(base, BPW)])
        pltpu.sync_copy(grows, out_g.at[pl.ds(base, BPW)])

    return k(uidx, bidx, gidx, gflat, user_table, book_table)


BLK = 2048


def _mlp_body(u_ref, b_ref, g_ref, W1_ref, b1_ref, W2_ref, b2_ref,
              W3_ref, b3_ref, W4_ref, b4_ref, out_ref):
    f32 = jnp.float32
    h = jnp.dot(u_ref[...], W1_ref[0:EMB, :], preferred_element_type=f32)
    h = h + jnp.dot(b_ref[...], W1_ref[EMB:2 * EMB, :],
                    preferred_element_type=f32)
    h = h + jnp.dot(g_ref[...], W1_ref[2 * EMB:, :],
                    preferred_element_type=f32)
    h = jnp.maximum(h + b1_ref[...], 0.0)
    h = jnp.maximum(
        jnp.dot(h, W2_ref[...], preferred_element_type=f32) + b2_ref[...], 0.0)
    h = jnp.maximum(
        jnp.dot(h, W3_ref[...], preferred_element_type=f32) + b3_ref[...], 0.0)
    out_ref[...] = (
        jnp.dot(h, W4_ref[...], preferred_element_type=f32) + b4_ref[...])


def _mlp(u, b, g, W1, b1, W2, b2, W3, b3, W4, b4):
    d_in = 2 * EMB + NGT * GEMB
    grid = B // BLK
    full = lambda shape: pl.BlockSpec(shape, lambda i: (0, 0))
    return pl.pallas_call(
        _mlp_body,
        grid=(grid,),
        in_specs=[
            pl.BlockSpec((BLK, EMB), lambda i: (i, 0)),
            pl.BlockSpec((BLK, EMB), lambda i: (i, 0)),
            pl.BlockSpec((BLK, NGT * GEMB), lambda i: (i, 0)),
            full((d_in, 32)),
            full((1, 32)),
            full((32, 16)),
            full((1, 16)),
            full((16, 8)),
            full((1, 8)),
            full((8, 1)),
            full((1, 1)),
        ],
        out_specs=pl.BlockSpec((BLK, 1), lambda i: (i, 0)),
        out_shape=jax.ShapeDtypeStruct((B, 1), jnp.float32),
    )(u, b, g, W1, b1.reshape(1, -1), W2, b2.reshape(1, -1),
      W3, b3.reshape(1, -1), W4, b4.reshape(1, -1))


def kernel(user_id, book_title,
           user_genre_cat_1, user_genre_cat_2, user_genre_cat_3,
           user_genre_cat_4,
           book_genre_cat_1, book_genre_cat_2, book_genre_cat_3,
           book_genre_cat_4,
           user_table, book_table,
           ug_table_1, ug_table_2, ug_table_3, ug_table_4,
           bg_table_1, bg_table_2, bg_table_3, bg_table_4,
           W1, b1, W2, b2, W3, b3, W4, b4):
    uidx = user_id.reshape(NW, NCHUNK, CH)
    bidx = book_title.reshape(NW, NCHUNK, CH)
    gids = jnp.stack([
        user_genre_cat_1, user_genre_cat_2, user_genre_cat_3,
        user_genre_cat_4,
        book_genre_cat_1, book_genre_cat_2, book_genre_cat_3,
        book_genre_cat_4,
    ])
    gidx = gids.reshape(NGT, NW, BPW).transpose(1, 0, 2)
    gflat = jnp.concatenate([
        ug_table_1.reshape(-1), ug_table_2.reshape(-1),
        ug_table_3.reshape(-1), ug_table_4.reshape(-1),
        bg_table_1.reshape(-1), bg_table_2.reshape(-1),
        bg_table_3.reshape(-1), bg_table_4.reshape(-1),
    ])

    u_rows, b_rows, g_rows = _sc_gather(uidx, bidx, gidx, gflat,
                                        user_table, book_table)
    return _mlp(u_rows, b_rows, g_rows, W1, b1, W2, b2, W3, b3, W4, b4)
